# Initial kernel scaffold; baseline (speedup 1.0000x reference)
import jax
import jax.numpy as jnp
from jax.experimental import pallas as pl


def kernel(g, h_in, e, H0, H1, H2, W0, W1, W2, W3, nW0, nb0, nW1, nb1, nW2, nb2, nW3, nb3):
    raise NotImplementedError("write your pallas kernel here")


# speedup vs baseline: 2.9648x; 2.9648x over previous
import functools

import jax
import jax.numpy as jnp
from jax.experimental import pallas as pl
from jax.experimental.pallas import tpu as pltpu

B, N, D_IN, D_E, OUT, TGT = 64, 32, 128, 16, 128, 12
NDEG = 33
P = B * N
MSG = D_IN + D_E
EW = N * D_E

_F32 = jnp.float32


def _dot(a, b):
    return jax.lax.dot_general(
        a, b, (((1,), (0,)), ((), ())), preferred_element_type=_F32)


def _gnn_kernel(g_ref, e_ref, h_ref, H0_ref, H1_ref, H2_ref,
                W0_ref, W1_ref, W2_ref, W3_ref,
                nW0_ref, nb0_ref, nW1_ref, nb1_ref,
                nW2_ref, nb2_ref, nW3_ref, nb3_ref,
                out_ref, gbd_ref):
    g = g_ref[...]
    deg = jnp.sum(g, axis=1, keepdims=True)
    deg = jnp.minimum(deg, float(NDEG - 1))

    tq = jax.lax.broadcasted_iota(jnp.int32, (N, P), 1)
    tw = jax.lax.broadcasted_iota(jnp.int32, (N, P), 0)
    T = (tq % N == tw).astype(_F32)
    CH = 256
    for c in range(P // CH):
        rows = _dot(g_ref[c * CH:(c + 1) * CH, :], T)
        ri = jax.lax.broadcasted_iota(jnp.int32, (CH, P), 0)
        ci = jax.lax.broadcasted_iota(jnp.int32, (CH, P), 1)
        blk = ((ri + c * CH) // N == ci // N).astype(_F32)
        gbd_ref[c * CH:(c + 1) * CH, :] = rows * blk

    rl = jax.lax.broadcasted_iota(jnp.int32, (N, EW), 1)
    rw = jax.lax.broadcasted_iota(jnp.int32, (N, EW), 0)
    R = (rl // D_E == rw).astype(_F32)
    sl = jax.lax.broadcasted_iota(jnp.int32, (EW, D_E), 0)
    sj = jax.lax.broadcasted_iota(jnp.int32, (EW, D_E), 1)
    S = (sl % D_E == sj).astype(_F32)
    g_rep = _dot(g, R)
    m_e = _dot(g_rep * e_ref[...], S)

    si = jax.lax.broadcasted_iota(jnp.int32, (B, P), 0)
    sp = jax.lax.broadcasted_iota(jnp.int32, (B, P), 1)
    Ssum = (sp // N == si).astype(_F32)

    def readout(h_l, W_ref):
        mask = (jnp.sum(h_l, axis=1, keepdims=True) != 0).astype(_F32)
        hsum = _dot(Ssum, h_l * mask)
        return _dot(hsum, W_ref[...])

    h = h_ref[...]
    aux = readout(h, W0_ref)

    for H_ref, W_ref in ((H0_ref, W1_ref), (H1_ref, W2_ref), (H2_ref, W3_ref)):
        m_h = _dot(gbd_ref[...], h)
        m = jnp.concatenate([m_h, m_e], axis=1)
        acc = jnp.zeros((P, OUT), dtype=_F32)
        for d in range(NDEG):
            acc = acc + jnp.where(deg == float(d), _dot(m, H_ref[d]), 0.0)
        h = jax.nn.sigmoid(acc)
        aux = aux + readout(h, W_ref)

    s = jax.nn.softmax(aux, axis=1)
    x = jax.nn.relu(_dot(s, nW0_ref[...]) + nb0_ref[...])
    x = jax.nn.relu(_dot(x, nW1_ref[...]) + nb1_ref[...])
    x = jax.nn.relu(_dot(x, nW2_ref[...]) + nb2_ref[...])
    x = jax.nn.sigmoid(_dot(x, nW3_ref[...]) + nb3_ref[...])
    out_ref[...] = jax.nn.softmax(x, axis=1)


@functools.partial(jax.jit, static_argnames=("interpret",))
def _run(g, h_in, e, H0, H1, H2, W0, W1, W2, W3,
         nW0, nb0, nW1, nb1, nW2, nb2, nW3, nb3, interpret=False):
    g2 = g.reshape(P, N)
    e2 = e.reshape(P, EW)
    h2 = h_in.reshape(P, D_IN)
    return pl.pallas_call(
        _gnn_kernel,
        out_shape=jax.ShapeDtypeStruct((B, TGT), _F32),
        scratch_shapes=[pltpu.VMEM((P, P), _F32)],
        interpret=interpret,
    )(g2, e2, h2, H0, H1, H2, W0, W1, W2, W3,
      nW0, nb0.reshape(1, -1), nW1, nb1.reshape(1, -1),
      nW2, nb2.reshape(1, -1), nW3, nb3.reshape(1, -1))


def kernel(g, h_in, e, H0, H1, H2, W0, W1, W2, W3,
           nW0, nb0, nW1, nb1, nW2, nb2, nW3, nb3):
    return _run(g, h_in, e, H0, H1, H2, W0, W1, W2, W3,
                nW0, nb0, nW1, nb1, nW2, nb2, nW3, nb3)
